# baseline (speedup 1.0000x reference)
import functools

import jax
import jax.numpy as jnp
from jax import lax
from jax.experimental import pallas as pl
from jax.experimental.pallas import tpu as pltpu
from jax.experimental.pallas import tpu_sc as plsc

NUM_NODES = 1000000
H_DIM = 16
BATCH = 16384

NC = 2
NS = 16
NW = NC * NS
B_PER_W = BATCH // NW
CHUNK = 128
N_CHUNKS = B_PER_W // CHUNK

_mesh = plsc.VectorSubcoreMesh(core_axis_name="c", subcore_axis_name="s")


@functools.partial(
    pl.kernel,
    mesh=_mesh,
    out_type=jax.ShapeDtypeStruct((BATCH, H_DIM), jnp.float32),
    scratch_types=[
        pltpu.VMEM((N_CHUNKS, CHUNK), jnp.int32),
        pltpu.VMEM((B_PER_W, H_DIM), jnp.float32),
        pltpu.SemaphoreType.DMA,
    ],
    compiler_params=pltpu.CompilerParams(use_tc_tiling_on_sc=False),
)
def _gather_kernel(table_hbm, idx_hbm, out_hbm, idx_v, rows_v, sem):
    wid = lax.axis_index("s") * NC + lax.axis_index("c")
    base = wid * B_PER_W
    pltpu.sync_copy(idx_hbm.at[wid], idx_v)
    copies = []
    for j in range(N_CHUNKS):
        copies.append(
            pltpu.async_copy(
                table_hbm.at[idx_v.at[j]],
                rows_v.at[pl.ds(j * CHUNK, CHUNK), :],
                sem,
            )
        )
    for c in copies:
        c.wait()
    pltpu.sync_copy(rows_v, out_hbm.at[pl.ds(base, B_PER_W)])


def kernel(g, h, r, norm, table):
    idx = jnp.reshape(h.astype(jnp.int32), (NW, N_CHUNKS, CHUNK))
    return _gather_kernel(table, idx)


# speedup vs baseline: 7.1097x; 7.1097x over previous
import functools

import jax
import jax.numpy as jnp
from jax import lax
from jax.experimental import pallas as pl
from jax.experimental.pallas import tpu as pltpu
from jax.experimental.pallas import tpu_sc as plsc

NUM_NODES = 1000000
H_DIM = 16
BATCH = 16384

NC = 2
NS = 16
NW = NC * NS
B_PER_W = BATCH // NW
K = 16
N_CHUNKS = B_PER_W // K

_mesh = plsc.VectorSubcoreMesh(core_axis_name="c", subcore_axis_name="s")


@functools.partial(
    pl.kernel,
    mesh=_mesh,
    out_type=jax.ShapeDtypeStruct((H_DIM, BATCH), jnp.float32),
    scratch_types=[
        pltpu.VMEM((B_PER_W,), jnp.int32),
        pltpu.VMEM((2 * K, H_DIM, 128), jnp.float32),
        pltpu.VMEM((H_DIM, B_PER_W), jnp.float32),
        pltpu.SemaphoreType.DMA,
    ],
    compiler_params=pltpu.CompilerParams(needs_layout_passes=False),
)
def _gather_kernel(tab_hbm, idx_hbm, out_hbm, idx_v, blocks_v, rows_v, sem):
    wid = lax.axis_index("s") * NC + lax.axis_index("c")
    base = wid * B_PER_W
    pltpu.sync_copy(idx_hbm.at[pl.ds(base, B_PER_W)], idx_v)

    lane_iota = lax.iota(jnp.int32, 16)

    def fire(g, slot):
        ivec = idx_v[pl.ds(g * K, K)]
        for k in range(K):
            i = ivec[k]
            c128 = pl.multiple_of((i >> 7) << 7, 128)
            pltpu.async_copy(
                tab_hbm.at[:, pl.ds(c128, 128)],
                blocks_v.at[slot * K + k],
                sem,
            )

    def drain_and_extract(g, slot):
        for k in range(K):
            pltpu.make_async_copy(
                tab_hbm.at[:, pl.ds(0, 128)],
                blocks_v.at[slot * K + k],
                sem,
            ).wait()
        ivec = idx_v[pl.ds(g * K, K)]
        lvec = lax.rem(ivec, 128)
        tvec = g * K + lane_iota
        for k in range(K):
            vals = plsc.load_gather(
                blocks_v.at[slot * K + k],
                [lane_iota, jnp.full((16,), 1, jnp.int32) * lvec[k]],
            )
            plsc.store_scatter(
                rows_v,
                [lane_iota, jnp.full((16,), 1, jnp.int32) * tvec[k]],
                vals,
            )

    def body(g, carry):
        slot = lax.rem(g, 2)

        @pl.when(g < N_CHUNKS)
        def _():
            fire(g, slot)

        drain_and_extract(g - 1, lax.rem(g + 1, 2))
        return carry

    fire(0, 0)
    lax.fori_loop(1, N_CHUNKS + 1, body, 0)

    pltpu.sync_copy(rows_v, out_hbm.at[:, pl.ds(base, B_PER_W)])


def kernel(g, h, r, norm, table):
    tab_t = jnp.transpose(table)
    idx = h.astype(jnp.int32)
    out_t = _gather_kernel(tab_t, idx)
    return jnp.transpose(out_t)
